# baseline (speedup 1.0000x reference)
import jax
import jax.numpy as jnp
import numpy as np
from jax.experimental import pallas as pl
from jax.experimental.pallas import tpu as pltpu

N_TOK = 2048
N_EMBED = 512
EMBED_DIM = 64
BLK = 1024
GRID = N_TOK // BLK


def _threefry2x32(k0, k1, x0, x1):
    x0 = np.asarray(x0, np.uint32).copy()
    x1 = np.asarray(x1, np.uint32).copy()
    ks0 = np.uint32(k0)
    ks1 = np.uint32(k1)
    ks2 = np.uint32(np.uint32(0x1BD11BDA) ^ ks0 ^ ks1)
    ks = [ks0, ks1, ks2]
    rot1 = (13, 15, 26, 6)
    rot2 = (17, 29, 16, 24)

    def rotl(v, r):
        return ((v << np.uint32(r)) | (v >> np.uint32(32 - r))).astype(np.uint32)

    with np.errstate(over="ignore"):
        x0 = (x0 + ks0).astype(np.uint32)
        x1 = (x1 + ks1).astype(np.uint32)
        for i in range(5):
            for r in rot1 if i % 2 == 0 else rot2:
                x0 = (x0 + x1).astype(np.uint32)
                x1 = rotl(x1, r)
                x1 = (x0 ^ x1).astype(np.uint32)
            x0 = (x0 + ks[(i + 1) % 3]).astype(np.uint32)
            x1 = (x1 + ks[(i + 2) % 3] + np.uint32(i + 1)).astype(np.uint32)
    return x0, x1


def _make_gumbel_consts():
    k0, k1 = _threefry2x32(0, 1234, [0], [7])
    n = N_TOK * N_EMBED
    cnt = np.arange(n, dtype=np.uint32)
    o0, o1 = _threefry2x32(k0[0], k1[0], np.zeros(n, np.uint32), cnt)
    bits = o0 ^ o1
    u = ((bits >> np.uint32(9)) | np.uint32(0x3F800000)).view(np.float32)
    u = u - np.float32(1.0)
    eps = np.float32(np.finfo(np.float32).eps)
    u = np.clip(u, eps, np.float32(1.0) - eps)
    g = (-np.log(-np.log(u, dtype=np.float32), dtype=np.float32)).astype(np.float32)
    c = np.exp(-g, dtype=np.float32).astype(np.float32)
    return g.reshape(N_TOK, N_EMBED), c.reshape(N_TOK, N_EMBED)


_GUMBELS_NP, _EXP_NEG_GUMBELS_NP = _make_gumbel_consts()


def _body(xt_ref, et_ref, lv_ref, g_ref, c_ref, quant_ref, idx_ref, loss_ref,
          perp_ref, en_ref, counts_ref, acc_ref, es_ref):
    i = pl.program_id(0)
    e2 = et_ref[...]
    prec = jnp.exp(-lv_ref[0])
    x2 = xt_ref[...]

    hi = jax.lax.Precision.HIGHEST

    @pl.when(i == 0)
    def _en():
        es_ref[...] = e2 * prec
        en_ref[...] = 0.5 * jax.lax.dot_general(
            jnp.ones((1, EMBED_DIM), jnp.float32), e2 * es_ref[...],
            (((1,), (0,)), ((), ())),
            preferred_element_type=jnp.float32, precision=hi)
        counts_ref[...] = jnp.zeros_like(counts_ref)
        acc_ref[...] = jnp.zeros_like(acc_ref)

    xe = jax.lax.dot_general(x2, es_ref[...], (((0,), (0,)), ((), ())),
                             preferred_element_type=jnp.float32,
                             precision=hi)
    lvar = xe - en_ref[...]

    lmax = jnp.max(lvar, axis=1, keepdims=True)
    iota = jax.lax.broadcasted_iota(jnp.int32, (BLK, N_EMBED), 1)
    idx = jnp.min(jnp.where(lvar >= lmax, iota, N_EMBED), axis=1)
    idx_ref[...] = idx

    lg = lvar + g_ref[...]
    m1 = jnp.max(lg, axis=1, keepdims=True)
    eu = jnp.exp(lg - m1)
    s1 = jnp.sum(eu, axis=1, keepdims=True)
    qu = jax.lax.dot_general(e2, eu, (((1,), (1,)), ((), ())),
                             preferred_element_type=jnp.float32,
                             precision=jax.lax.Precision.DEFAULT)
    quant = qu * jnp.reshape(1.0 / s1, (1, BLK))
    quant_ref[...] = quant

    pc = eu * c_ref[...]
    s2 = jnp.sum(pc, axis=1, keepdims=True)
    spl = jnp.sum(pc * lvar, axis=1, keepdims=True)
    plogp = jnp.sum(spl / s2 - m1 - jnp.log(s2))

    sq = jnp.sum((x2 - quant) ** 2)
    block_loss = 0.5 * prec * sq + plogp

    cnt = jnp.sum(jnp.where(idx.reshape(BLK, 1) == iota, 1.0, 0.0),
                  axis=0, keepdims=True)

    acc_ref[...] += block_loss.reshape(1, 1)
    counts_ref[...] += cnt

    @pl.when(i == GRID - 1)
    def _finish():
        loss_ref[...] = acc_ref[0, :] + jnp.zeros((1,), jnp.float32)
        avg = counts_ref[...] * (1.0 / N_TOK)
        perp_ref[...] = jnp.exp(-jnp.sum(avg * jnp.log(avg + 1e-10),
                                         axis=1))


def kernel(x, embedding, log_var_q_scalar):
    g = jnp.asarray(_GUMBELS_NP)
    c = jnp.asarray(_EXP_NEG_GUMBELS_NP)

    quant_t, idx, loss, perp = pl.pallas_call(
        _body,
        grid=(GRID,),
        in_specs=[
            pl.BlockSpec((EMBED_DIM, BLK), lambda i: (0, i)),
            pl.BlockSpec((EMBED_DIM, N_EMBED), lambda i: (0, 0)),
            pl.BlockSpec(memory_space=pltpu.SMEM),
            pl.BlockSpec((BLK, N_EMBED), lambda i: (i, 0)),
            pl.BlockSpec((BLK, N_EMBED), lambda i: (i, 0)),
        ],
        out_specs=[
            pl.BlockSpec((EMBED_DIM, BLK), lambda i: (0, i)),
            pl.BlockSpec((BLK,), lambda i: (i,)),
            pl.BlockSpec((1,), lambda i: (0,)),
            pl.BlockSpec((1,), lambda i: (0,)),
        ],
        out_shape=[
            jax.ShapeDtypeStruct((EMBED_DIM, N_TOK), jnp.float32),
            jax.ShapeDtypeStruct((N_TOK,), jnp.int32),
            jax.ShapeDtypeStruct((1,), jnp.float32),
            jax.ShapeDtypeStruct((1,), jnp.float32),
        ],
        scratch_shapes=[
            pltpu.VMEM((1, N_EMBED), jnp.float32),
            pltpu.VMEM((1, N_EMBED), jnp.float32),
            pltpu.VMEM((1, 1), jnp.float32),
            pltpu.VMEM((EMBED_DIM, N_EMBED), jnp.float32),
        ],
    )(x.T, embedding.T, log_var_q_scalar, g, c)

    return quant_t.T, idx, loss.reshape(()), perp.reshape(())


# speedup vs baseline: 1.0763x; 1.0763x over previous
import jax
import jax.numpy as jnp
import numpy as np
from jax.experimental import pallas as pl
from jax.experimental.pallas import tpu as pltpu

N_TOK = 2048
N_EMBED = 512
EMBED_DIM = 64
BLK = 1024
GRID = N_TOK // BLK


def _threefry2x32(k0, k1, x0, x1):
    x0 = np.asarray(x0, np.uint32).copy()
    x1 = np.asarray(x1, np.uint32).copy()
    ks0 = np.uint32(k0)
    ks1 = np.uint32(k1)
    ks2 = np.uint32(np.uint32(0x1BD11BDA) ^ ks0 ^ ks1)
    ks = [ks0, ks1, ks2]
    rot1 = (13, 15, 26, 6)
    rot2 = (17, 29, 16, 24)

    def rotl(v, r):
        return ((v << np.uint32(r)) | (v >> np.uint32(32 - r))).astype(np.uint32)

    with np.errstate(over="ignore"):
        x0 = (x0 + ks0).astype(np.uint32)
        x1 = (x1 + ks1).astype(np.uint32)
        for i in range(5):
            for r in rot1 if i % 2 == 0 else rot2:
                x0 = (x0 + x1).astype(np.uint32)
                x1 = rotl(x1, r)
                x1 = (x0 ^ x1).astype(np.uint32)
            x0 = (x0 + ks[(i + 1) % 3]).astype(np.uint32)
            x1 = (x1 + ks[(i + 2) % 3] + np.uint32(i + 1)).astype(np.uint32)
    return x0, x1


def _make_gumbel_consts():
    k0, k1 = _threefry2x32(0, 1234, [0], [7])
    n = N_TOK * N_EMBED
    cnt = np.arange(n, dtype=np.uint32)
    o0, o1 = _threefry2x32(k0[0], k1[0], np.zeros(n, np.uint32), cnt)
    bits = o0 ^ o1
    u = ((bits >> np.uint32(9)) | np.uint32(0x3F800000)).view(np.float32)
    u = u - np.float32(1.0)
    eps = np.float32(np.finfo(np.float32).eps)
    u = np.clip(u, eps, np.float32(1.0) - eps)
    g = (-np.log(-np.log(u, dtype=np.float32), dtype=np.float32)).astype(np.float32)
    c = np.exp(-g, dtype=np.float32).astype(np.float32)
    return g.reshape(N_TOK, N_EMBED), c.reshape(N_TOK, N_EMBED)


_GUMBELS_NP, _EXP_NEG_GUMBELS_NP = _make_gumbel_consts()


def _body(xt_ref, et_ref, lv_ref, g_ref, quant_ref, idx_ref, loss_ref,
          perp_ref, en_ref, counts_ref, acc_ref, es_ref):
    i = pl.program_id(0)
    e2 = et_ref[...]
    prec = jnp.exp(-lv_ref[0])
    x2 = xt_ref[...]

    hi = jax.lax.Precision.HIGHEST

    @pl.when(i == 0)
    def _en():
        es_ref[...] = e2 * prec
        en_ref[...] = 0.5 * jax.lax.dot_general(
            jnp.ones((1, EMBED_DIM), jnp.float32), e2 * es_ref[...],
            (((1,), (0,)), ((), ())),
            preferred_element_type=jnp.float32, precision=hi)
        counts_ref[...] = jnp.zeros_like(counts_ref)
        acc_ref[...] = jnp.zeros_like(acc_ref)

    xe = jax.lax.dot_general(x2, es_ref[...], (((0,), (0,)), ((), ())),
                             preferred_element_type=jnp.float32,
                             precision=hi)
    lvar = xe - en_ref[...]

    lmax = jnp.max(lvar, axis=1, keepdims=True)
    iota = jax.lax.broadcasted_iota(jnp.int32, (BLK, N_EMBED), 1)
    idx = jnp.min(jnp.where(lvar >= lmax, iota, N_EMBED), axis=1)
    idx_ref[...] = idx

    lg = lvar + g_ref[...]
    m1 = jnp.max(lg, axis=1, keepdims=True)
    eu = jnp.exp(lg - m1)
    s1 = jnp.sum(eu, axis=1, keepdims=True)
    qu = jax.lax.dot_general(e2, eu, (((1,), (1,)), ((), ())),
                             preferred_element_type=jnp.float32,
                             precision=jax.lax.Precision.DEFAULT)
    quant = qu * jnp.reshape(1.0 / s1, (1, BLK))
    quant_ref[...] = quant

    pc = jnp.exp(lvar - m1)
    s2 = jnp.sum(pc, axis=1, keepdims=True)
    spl = jnp.sum(pc * lvar, axis=1, keepdims=True)
    plogp = jnp.sum(spl / s2 - m1 - jnp.log(s2))

    sq = jnp.sum((x2 - quant) ** 2)
    block_loss = 0.5 * prec * sq + plogp

    cnt = jnp.sum(jnp.where(idx.reshape(BLK, 1) == iota, 1.0, 0.0),
                  axis=0, keepdims=True)

    acc_ref[...] += block_loss.reshape(1, 1)
    counts_ref[...] += cnt

    @pl.when(i == GRID - 1)
    def _finish():
        loss_ref[...] = acc_ref[0, :] + jnp.zeros((1,), jnp.float32)
        avg = counts_ref[...] * (1.0 / N_TOK)
        perp_ref[...] = jnp.exp(-jnp.sum(avg * jnp.log(avg + 1e-10),
                                         axis=1))


def kernel(x, embedding, log_var_q_scalar):
    g = jnp.asarray(_GUMBELS_NP)

    quant_t, idx, loss, perp = pl.pallas_call(
        _body,
        grid=(GRID,),
        in_specs=[
            pl.BlockSpec((EMBED_DIM, BLK), lambda i: (0, i)),
            pl.BlockSpec((EMBED_DIM, N_EMBED), lambda i: (0, 0)),
            pl.BlockSpec(memory_space=pltpu.SMEM),
            pl.BlockSpec((BLK, N_EMBED), lambda i: (i, 0)),
        ],
        out_specs=[
            pl.BlockSpec((EMBED_DIM, BLK), lambda i: (0, i)),
            pl.BlockSpec((BLK,), lambda i: (i,)),
            pl.BlockSpec((1,), lambda i: (0,)),
            pl.BlockSpec((1,), lambda i: (0,)),
        ],
        out_shape=[
            jax.ShapeDtypeStruct((EMBED_DIM, N_TOK), jnp.float32),
            jax.ShapeDtypeStruct((N_TOK,), jnp.int32),
            jax.ShapeDtypeStruct((1,), jnp.float32),
            jax.ShapeDtypeStruct((1,), jnp.float32),
        ],
        scratch_shapes=[
            pltpu.VMEM((1, N_EMBED), jnp.float32),
            pltpu.VMEM((1, N_EMBED), jnp.float32),
            pltpu.VMEM((1, 1), jnp.float32),
            pltpu.VMEM((EMBED_DIM, N_EMBED), jnp.float32),
        ],
    )(x.T, embedding.T, log_var_q_scalar, g)

    return quant_t.T, idx, loss.reshape(()), perp.reshape(())


# speedup vs baseline: 1.0796x; 1.0031x over previous
import jax
import jax.numpy as jnp
import numpy as np
from jax.experimental import pallas as pl
from jax.experimental.pallas import tpu as pltpu

N_TOK = 2048
N_EMBED = 512
EMBED_DIM = 64
BLK = 1024
GRID = N_TOK // BLK


def _threefry2x32(k0, k1, x0, x1):
    x0 = np.asarray(x0, np.uint32).copy()
    x1 = np.asarray(x1, np.uint32).copy()
    ks0 = np.uint32(k0)
    ks1 = np.uint32(k1)
    ks2 = np.uint32(np.uint32(0x1BD11BDA) ^ ks0 ^ ks1)
    ks = [ks0, ks1, ks2]
    rot1 = (13, 15, 26, 6)
    rot2 = (17, 29, 16, 24)

    def rotl(v, r):
        return ((v << np.uint32(r)) | (v >> np.uint32(32 - r))).astype(np.uint32)

    with np.errstate(over="ignore"):
        x0 = (x0 + ks0).astype(np.uint32)
        x1 = (x1 + ks1).astype(np.uint32)
        for i in range(5):
            for r in rot1 if i % 2 == 0 else rot2:
                x0 = (x0 + x1).astype(np.uint32)
                x1 = rotl(x1, r)
                x1 = (x0 ^ x1).astype(np.uint32)
            x0 = (x0 + ks[(i + 1) % 3]).astype(np.uint32)
            x1 = (x1 + ks[(i + 2) % 3] + np.uint32(i + 1)).astype(np.uint32)
    return x0, x1


def _make_gumbels():
    k0, k1 = _threefry2x32(0, 1234, [0], [7])
    n = N_TOK * N_EMBED
    cnt = np.arange(n, dtype=np.uint32)
    o0, o1 = _threefry2x32(k0[0], k1[0], np.zeros(n, np.uint32), cnt)
    bits = o0 ^ o1
    u = ((bits >> np.uint32(9)) | np.uint32(0x3F800000)).view(np.float32)
    u = u - np.float32(1.0)
    eps = np.float32(np.finfo(np.float32).eps)
    u = np.clip(u, eps, np.float32(1.0) - eps)
    g = (-np.log(-np.log(u, dtype=np.float32), dtype=np.float32)).astype(np.float32)
    return g.reshape(N_TOK, N_EMBED)


_GUMBELS_NP = _make_gumbels()


def _body(xt_ref, et_ref, lv_ref, g_ref, quant_ref, idx_ref, loss_ref,
          perp_ref, en_ref, counts_ref, acc_ref, es_ref):
    i = pl.program_id(0)
    e2 = et_ref[...]
    prec = jnp.exp(-lv_ref[0])
    x2 = xt_ref[...]

    hi = jax.lax.Precision.HIGHEST

    @pl.when(i == 0)
    def _en():
        es_ref[...] = e2 * prec
        en_ref[...] = 0.5 * jax.lax.dot_general(
            jnp.ones((1, EMBED_DIM), jnp.float32), e2 * es_ref[...],
            (((1,), (0,)), ((), ())),
            preferred_element_type=jnp.float32, precision=hi)
        counts_ref[...] = jnp.zeros_like(counts_ref)
        acc_ref[...] = jnp.zeros_like(acc_ref)

    xe = jax.lax.dot_general(x2, es_ref[...], (((0,), (0,)), ((), ())),
                             preferred_element_type=jnp.float32,
                             precision=hi)
    lvar = xe - en_ref[...]

    lmax = jnp.max(lvar, axis=1, keepdims=True)
    iota = jax.lax.broadcasted_iota(jnp.int32, (BLK, N_EMBED), 1)
    idx = jnp.min(jnp.where(lvar >= lmax, iota, N_EMBED), axis=1)
    idx_ref[...] = idx

    lg = lvar + g_ref[...]
    m1 = jnp.max(lg, axis=1, keepdims=True)
    eu = jnp.exp(lg - m1)
    s1 = jnp.sum(eu, axis=1, keepdims=True)
    qu = jax.lax.dot_general(e2, eu, (((1,), (1,)), ((), ())),
                             preferred_element_type=jnp.float32,
                             precision=jax.lax.Precision.DEFAULT)
    quant = qu * jnp.reshape(1.0 / s1, (1, BLK))
    quant_ref[...] = quant

    pc = jnp.exp(lvar - m1)
    s2 = jnp.sum(pc, axis=1, keepdims=True)
    spl = jnp.sum(pc * lvar, axis=1, keepdims=True)
    plogp = jnp.sum(spl / s2 - m1 - jnp.log(s2))

    sq = jnp.sum((x2 - quant) ** 2)
    block_loss = 0.5 * prec * sq + plogp

    cnt = jnp.sum(jnp.where(idx.reshape(BLK, 1) == iota, 1.0, 0.0),
                  axis=0, keepdims=True)

    acc_ref[...] += block_loss.reshape(1, 1)
    counts_ref[...] += cnt

    @pl.when(i == GRID - 1)
    def _finish():
        loss_ref[...] = acc_ref[0, :] + jnp.zeros((1,), jnp.float32)
        avg = counts_ref[...] * (1.0 / N_TOK)
        perp_ref[...] = jnp.exp(-jnp.sum(avg * jnp.log(avg + 1e-10),
                                         axis=1))


def kernel(x, embedding, log_var_q_scalar):
    g = jnp.asarray(_GUMBELS_NP)

    quant_t, idx, loss, perp = pl.pallas_call(
        _body,
        grid=(GRID,),
        in_specs=[
            pl.BlockSpec((EMBED_DIM, BLK), lambda i: (0, i)),
            pl.BlockSpec((EMBED_DIM, N_EMBED), lambda i: (0, 0)),
            pl.BlockSpec(memory_space=pltpu.SMEM),
            pl.BlockSpec((BLK, N_EMBED), lambda i: (i, 0)),
        ],
        out_specs=[
            pl.BlockSpec((EMBED_DIM, BLK), lambda i: (0, i)),
            pl.BlockSpec((BLK,), lambda i: (i,)),
            pl.BlockSpec((1,), lambda i: (0,)),
            pl.BlockSpec((1,), lambda i: (0,)),
        ],
        out_shape=[
            jax.ShapeDtypeStruct((EMBED_DIM, N_TOK), jnp.float32),
            jax.ShapeDtypeStruct((N_TOK,), jnp.int32),
            jax.ShapeDtypeStruct((1,), jnp.float32),
            jax.ShapeDtypeStruct((1,), jnp.float32),
        ],
        scratch_shapes=[
            pltpu.VMEM((1, N_EMBED), jnp.float32),
            pltpu.VMEM((1, N_EMBED), jnp.float32),
            pltpu.VMEM((1, 1), jnp.float32),
            pltpu.VMEM((EMBED_DIM, N_EMBED), jnp.float32),
        ],
    )(x.T, embedding.T, log_var_q_scalar, g)

    return quant_t.T, idx, loss.reshape(()), perp.reshape(())
